# Initial kernel scaffold; baseline (speedup 1.0000x reference)
import functools

import jax
import jax.numpy as jnp
from jax import lax
from jax.experimental import pallas as pl
from jax.experimental.pallas import tpu as pltpu
from jax.experimental.pallas import tpu_sc as plsc

_NC = 2
_NS = 16
_L = 16


@functools.cache
def _make_sc_agg(N, E, D, with_deg):
    NW = _NC * _NS
    EPW = E // NW
    K = 80
    NCH = EPW // K
    assert EPW % K == 0 and (EPW * NW) == E
    RPT = N // _NS
    ZR = 125
    assert N % _NS == 0 and RPT % ZR == 0
    DW = _L

    mesh = plsc.VectorSubcoreMesh(core_axis_name="c", subcore_axis_name="s",
                                  num_cores=_NC, num_subcores=_NS)

    out_type = [jax.ShapeDtypeStruct((_NC, N, D), jnp.float32)]
    scratch = [
        pltpu.VMEM((K,), jnp.int32),
        pltpu.VMEM((K,), jnp.int32),
        pltpu.VMEM((K, D), jnp.float32),
        pltpu.VMEM((ZR, D), jnp.float32),
        pltpu.VMEM_SHARED((N, D), jnp.float32),
        pltpu.SemaphoreType.DMA,
    ]
    if with_deg:
        out_type.append(jax.ShapeDtypeStruct((_NC, N, DW), jnp.float32))
        scratch += [
            pltpu.VMEM((K, DW), jnp.float32),
            pltpu.VMEM((ZR, DW), jnp.float32),
            pltpu.VMEM_SHARED((N, DW), jnp.float32),
        ]

    def body(*refs):
        if with_deg:
            (m_hbm, src_hbm, dst_hbm, agg_out, deg_out,
             srcv, dstv, rows, zbuf, aggs, sem, ones, zdeg, degs) = refs
        else:
            (m_hbm, src_hbm, dst_hbm, agg_out,
             srcv, dstv, rows, zbuf, aggs, sem) = refs
        c = lax.axis_index("c")
        s = lax.axis_index("s")
        wid = c * _NS + s

        zv = jnp.zeros((_L,), jnp.float32)

        @pl.loop(0, ZR)
        def _zero_zbuf(i):
            for j in range(D // _L):
                zbuf[i, pl.ds(j * _L, _L)] = zv

        r0 = s * RPT

        @pl.loop(0, RPT // ZR)
        def _zero_agg(t):
            pltpu.sync_copy(zbuf, aggs.at[pl.ds(r0 + t * ZR, ZR)])

        if with_deg:
            ov = jnp.full((_L,), 1.0, jnp.float32)

            @pl.loop(0, K)
            def _init_ones(i):
                ones[i, :] = ov

            @pl.loop(0, ZR)
            def _zero_zdeg(i):
                zdeg[i, :] = zv

            @pl.loop(0, RPT // ZR)
            def _zero_deg(t):
                pltpu.sync_copy(zdeg, degs.at[pl.ds(r0 + t * ZR, ZR)])

        plsc.subcore_barrier()

        ebase = wid * EPW

        @pl.loop(0, NCH)
        def _edges(chk):
            off = ebase + chk * K
            pltpu.sync_copy(src_hbm.at[pl.ds(off, K)], srcv)
            pltpu.sync_copy(dst_hbm.at[pl.ds(off, K)], dstv)
            pltpu.async_copy(m_hbm.at[srcv], rows, sem).wait()
            pltpu.sync_copy(rows, aggs.at[dstv], add=True)
            if with_deg:
                pltpu.sync_copy(ones, degs.at[dstv], add=True)

        plsc.subcore_barrier()

        pltpu.sync_copy(aggs.at[pl.ds(r0, RPT)], agg_out.at[c, pl.ds(r0, RPT)])
        if with_deg:
            pltpu.sync_copy(degs.at[pl.ds(r0, RPT)],
                            deg_out.at[c, pl.ds(r0, RPT)])

    return pl.kernel(body, out_type=out_type, mesh=mesh,
                     scratch_types=scratch)


def _mm_body(h_ref, w_ref, o_ref):
    o_ref[...] = jnp.dot(h_ref[...], w_ref[...],
                         preferred_element_type=jnp.float32)


@functools.cache
def _make_mm(N, D, BM):
    grid = (N // BM,)
    return pl.pallas_call(
        _mm_body,
        grid=grid,
        in_specs=[
            pl.BlockSpec((BM, D), lambda i: (i, 0)),
            pl.BlockSpec((D, D), lambda i: (0, 0)),
        ],
        out_specs=pl.BlockSpec((BM, D), lambda i: (i, 0)),
        out_shape=jax.ShapeDtypeStruct((N, D), jnp.float32),
    )


def _gru_core(aggp_ref, degp_ref, h_ref, wih_ref, whh_ref, bih_ref, bhh_ref):
    D = h_ref.shape[-1]
    deg = degp_ref[0, :, 0:1] + degp_ref[1, :, 0:1]
    deg = jnp.maximum(deg, 1.0)
    agg = (aggp_ref[0] + aggp_ref[1]) / deg
    h = h_ref[...]
    gi = jnp.dot(agg, wih_ref[...], preferred_element_type=jnp.float32)
    gi = gi + bih_ref[...]
    gh = jnp.dot(h, whh_ref[...], preferred_element_type=jnp.float32)
    gh = gh + bhh_ref[...]
    r = jax.nn.sigmoid(gi[:, :D] + gh[:, :D])
    z = jax.nn.sigmoid(gi[:, D:2 * D] + gh[:, D:2 * D])
    n = jnp.tanh(gi[:, 2 * D:] + r * gh[:, 2 * D:])
    return (1.0 - z) * n + z * h


def _gru_next_body(aggp_ref, degp_ref, h_ref, wih_ref, whh_ref, bih_ref,
                   bhh_ref, wnext_ref, hout_ref, mout_ref):
    hn = _gru_core(aggp_ref, degp_ref, h_ref, wih_ref, whh_ref, bih_ref,
                   bhh_ref)
    hout_ref[...] = hn
    mout_ref[...] = jnp.dot(hn, wnext_ref[...],
                            preferred_element_type=jnp.float32)


def _gru_final_body(aggp_ref, degp_ref, h_ref, wih_ref, whh_ref, bih_ref,
                    bhh_ref, out_ref):
    hn = _gru_core(aggp_ref, degp_ref, h_ref, wih_ref, whh_ref, bih_ref,
                   bhh_ref)
    out_ref[...] = jnp.tanh(jnp.mean(hn, axis=1, keepdims=True))


@functools.cache
def _make_gru(N, D, DW, BM, final):
    grid = (N // BM,)
    in_specs = [
        pl.BlockSpec((2, BM, D), lambda i: (0, i, 0)),
        pl.BlockSpec((2, BM, DW), lambda i: (0, i, 0)),
        pl.BlockSpec((BM, D), lambda i: (i, 0)),
        pl.BlockSpec((D, 3 * D), lambda i: (0, 0)),
        pl.BlockSpec((D, 3 * D), lambda i: (0, 0)),
        pl.BlockSpec((1, 3 * D), lambda i: (0, 0)),
        pl.BlockSpec((1, 3 * D), lambda i: (0, 0)),
    ]
    if final:
        return pl.pallas_call(
            _gru_final_body,
            grid=grid,
            in_specs=in_specs,
            out_specs=pl.BlockSpec((BM, 1), lambda i: (i, 0)),
            out_shape=jax.ShapeDtypeStruct((N, 1), jnp.float32),
        )
    in_specs.append(pl.BlockSpec((D, D), lambda i: (0, 0)))
    return pl.pallas_call(
        _gru_next_body,
        grid=grid,
        in_specs=in_specs,
        out_specs=[pl.BlockSpec((BM, D), lambda i: (i, 0)),
                   pl.BlockSpec((BM, D), lambda i: (i, 0))],
        out_shape=[jax.ShapeDtypeStruct((N, D), jnp.float32),
                   jax.ShapeDtypeStruct((N, D), jnp.float32)],
    )


def kernel(x, edge_index, weight, W_ih, W_hh, b_ih, b_hh):
    N, D = x.shape
    E = edge_index.shape[1]
    num_layers = weight.shape[0]
    BM = 1250
    DW = _L

    src = edge_index[0]
    dst = edge_index[1]
    W_ihT = W_ih.T
    W_hhT = W_hh.T
    b_ih2 = b_ih.reshape(1, -1)
    b_hh2 = b_hh.reshape(1, -1)

    sc_agg_deg = _make_sc_agg(N, E, D, True)
    sc_agg = _make_sc_agg(N, E, D, False)
    mm = _make_mm(N, D, BM)
    gru_next = _make_gru(N, D, DW, BM, False)
    gru_final = _make_gru(N, D, DW, BM, True)

    h = x
    m = mm(h, weight[0])
    degp = None
    for i in range(num_layers):
        if i == 0:
            aggp, degp = sc_agg_deg(m, src, dst)
        else:
            (aggp,) = sc_agg(m, src, dst)
        if i + 1 < num_layers:
            h, m = gru_next(aggp, degp, h, W_ihT, W_hhT, b_ih2, b_hh2,
                            weight[i + 1])
        else:
            out = gru_final(aggp, degp, h, W_ihT, W_hhT, b_ih2, b_hh2)
    return out


# speedup vs baseline: 4.9493x; 4.9493x over previous
import functools

import jax
import jax.numpy as jnp
from jax import lax
from jax.experimental import pallas as pl
from jax.experimental.pallas import tpu as pltpu
from jax.experimental.pallas import tpu_sc as plsc

_NC = 2
_NS = 16
_NW = _NC * _NS
_L = 16


def _sc_mesh():
    return plsc.VectorSubcoreMesh(core_axis_name="c", subcore_axis_name="s",
                                  num_cores=_NC, num_subcores=_NS)


@functools.cache
def _make_sc_agg(N, E, D):
    EPW = E // _NW
    K = 80
    NCH = EPW // K
    assert EPW % K == 0 and (EPW * _NW) == E
    CW = 10
    RPT = N // CW
    ZR = 40
    assert N % CW == 0 and RPT % ZR == 0 and ZR % 8 == 0 and K >= ZR

    @functools.partial(
        pl.kernel,
        out_type=jax.ShapeDtypeStruct((_NC, N, D), jnp.float32),
        mesh=_sc_mesh(),
        scratch_types=[
            pltpu.VMEM((K,), jnp.int32),
            pltpu.VMEM((K,), jnp.int32),
            pltpu.VMEM((K, D), jnp.float32),
            pltpu.VMEM_SHARED((N, D), jnp.float32),
            pltpu.SemaphoreType.DMA,
        ])
    def sc_agg(m_hbm, src_hbm, dst_hbm, agg_out, srcv, dstv, rows, aggs, sem):
        c = lax.axis_index("c")
        s = lax.axis_index("s")
        wid = c * _NS + s
        zv = jnp.zeros((_L,), jnp.float32)
        r0 = s * RPT

        @pl.when(s < CW)
        def _init():
            @pl.loop(0, ZR)
            def _zero_rows(i):
                for j in range(D // _L):
                    rows[i, pl.ds(j * _L, _L)] = zv

            @pl.loop(0, RPT // ZR)
            def _zero_agg(t):
                pltpu.sync_copy(rows.at[pl.ds(0, ZR)],
                                aggs.at[pl.ds(r0 + t * ZR, ZR)])

        plsc.subcore_barrier()

        ebase = wid * EPW

        @pl.loop(0, NCH)
        def _edges(chk):
            off = ebase + chk * K
            pltpu.sync_copy(src_hbm.at[pl.ds(off, K)], srcv)
            pltpu.sync_copy(dst_hbm.at[pl.ds(off, K)], dstv)
            pltpu.async_copy(m_hbm.at[srcv], rows, sem).wait()
            pltpu.sync_copy(rows, aggs.at[dstv], add=True)

        plsc.subcore_barrier()

        @pl.when(s < CW)
        def _copy_out():
            @pl.loop(0, RPT // ZR)
            def _copy_agg(t):
                rr = r0 + t * ZR
                pltpu.sync_copy(aggs.at[pl.ds(rr, ZR)], rows.at[pl.ds(0, ZR)])
                pltpu.sync_copy(rows.at[pl.ds(0, ZR)],
                                agg_out.at[c, pl.ds(rr, ZR)])

    return sc_agg


@functools.cache
def _make_sc_deg(N, E):
    EPW = E // _NW
    K = 80
    NCH = EPW // K
    assert EPW % K == 0 and N % _L == 0

    @functools.partial(
        pl.kernel,
        out_type=jax.ShapeDtypeStruct((_NW * N,), jnp.float32),
        mesh=_sc_mesh(),
        compiler_params=pltpu.CompilerParams(needs_layout_passes=False),
        scratch_types=[
            pltpu.VMEM((K,), jnp.int32),
            pltpu.VMEM((N,), jnp.float32),
        ])
    def sc_deg(dst_hbm, out_hbm, dstv, hist):
        c = lax.axis_index("c")
        s = lax.axis_index("s")
        wid = c * _NS + s
        zv = jnp.zeros((_L,), jnp.float32)
        ov = jnp.full((_L,), 1.0, jnp.float32)

        @pl.loop(0, N // _L)
        def _zero(i):
            hist[pl.ds(i * _L, _L)] = zv

        ebase = wid * EPW

        @pl.loop(0, NCH)
        def _edges(chk):
            off = ebase + chk * K
            pltpu.sync_copy(dst_hbm.at[pl.ds(off, K)], dstv)
            for j in range(K // _L):
                idx = dstv[pl.ds(j * _L, _L)]
                plsc.addupdate_scatter(hist, [idx], ov)

        pltpu.sync_copy(hist, out_hbm.at[pl.ds(wid * N, N)])

    return sc_deg


def _degsum_body(h_ref, o_ref):
    o_ref[...] = jnp.maximum(jnp.sum(h_ref[...], axis=0, keepdims=True), 1.0)


@functools.cache
def _make_degsum(N, BM):
    del BM
    return pl.pallas_call(
        _degsum_body,
        out_shape=jax.ShapeDtypeStruct((1, N), jnp.float32),
    )


def _mm_body(h_ref, w_ref, o_ref):
    o_ref[...] = jnp.dot(h_ref[...], w_ref[...],
                         preferred_element_type=jnp.float32)


@functools.cache
def _make_mm(N, D, BM):
    return pl.pallas_call(
        _mm_body,
        grid=(N // BM,),
        in_specs=[
            pl.BlockSpec((BM, D), lambda i: (i, 0)),
            pl.BlockSpec((D, D), lambda i: (0, 0)),
        ],
        out_specs=pl.BlockSpec((BM, D), lambda i: (i, 0)),
        out_shape=jax.ShapeDtypeStruct((N, D), jnp.float32),
    )


def _gru_cell(agg, h, wih_ref, whh_ref, bih_ref, bhh_ref):
    D = h.shape[-1]
    gi = jnp.dot(agg, wih_ref[...], preferred_element_type=jnp.float32)
    gi = gi + bih_ref[...]
    gh = jnp.dot(h, whh_ref[...], preferred_element_type=jnp.float32)
    gh = gh + bhh_ref[...]
    r = jax.nn.sigmoid(gi[:, :D] + gh[:, :D])
    z = jax.nn.sigmoid(gi[:, D:2 * D] + gh[:, D:2 * D])
    n = jnp.tanh(gi[:, 2 * D:] + r * gh[:, 2 * D:])
    return (1.0 - z) * n + z * h


def _gru_next_body(aggp_ref, deg_ref, h_ref, wih_ref, whh_ref, bih_ref,
                   bhh_ref, wnext_ref, hout_ref, mout_ref):
    agg = (aggp_ref[0] + aggp_ref[1]) / deg_ref[...]
    hn = _gru_cell(agg, h_ref[...], wih_ref, whh_ref, bih_ref, bhh_ref)
    hout_ref[...] = hn
    mout_ref[...] = jnp.dot(hn, wnext_ref[...],
                            preferred_element_type=jnp.float32)


def _gru_final_body(aggp_ref, deg_ref, h_ref, wih_ref, whh_ref, bih_ref,
                    bhh_ref, out_ref):
    agg = (aggp_ref[0] + aggp_ref[1]) / deg_ref[...]
    hn = _gru_cell(agg, h_ref[...], wih_ref, whh_ref, bih_ref, bhh_ref)
    out_ref[...] = jnp.tanh(jnp.mean(hn, axis=1, keepdims=True))


@functools.cache
def _make_gru(N, D, BM, final):
    in_specs = [
        pl.BlockSpec((2, BM, D), lambda i: (0, i, 0)),
        pl.BlockSpec((BM, 1), lambda i: (i, 0)),
        pl.BlockSpec((BM, D), lambda i: (i, 0)),
        pl.BlockSpec((D, 3 * D), lambda i: (0, 0)),
        pl.BlockSpec((D, 3 * D), lambda i: (0, 0)),
        pl.BlockSpec((1, 3 * D), lambda i: (0, 0)),
        pl.BlockSpec((1, 3 * D), lambda i: (0, 0)),
    ]
    if final:
        return pl.pallas_call(
            _gru_final_body,
            grid=(N // BM,),
            in_specs=in_specs,
            out_specs=pl.BlockSpec((BM, 1), lambda i: (i, 0)),
            out_shape=jax.ShapeDtypeStruct((N, 1), jnp.float32),
        )
    in_specs.append(pl.BlockSpec((D, D), lambda i: (0, 0)))
    return pl.pallas_call(
        _gru_next_body,
        grid=(N // BM,),
        in_specs=in_specs,
        out_specs=[pl.BlockSpec((BM, D), lambda i: (i, 0)),
                   pl.BlockSpec((BM, D), lambda i: (i, 0))],
        out_shape=[jax.ShapeDtypeStruct((N, D), jnp.float32),
                   jax.ShapeDtypeStruct((N, D), jnp.float32)],
    )


def kernel(x, edge_index, weight, W_ih, W_hh, b_ih, b_hh):
    N, D = x.shape
    E = edge_index.shape[1]
    num_layers = weight.shape[0]
    BM = 2000

    src = edge_index[0]
    dst = edge_index[1]
    W_ihT = W_ih.T
    W_hhT = W_hh.T
    b_ih2 = b_ih.reshape(1, -1)
    b_hh2 = b_hh.reshape(1, -1)

    sc_agg = _make_sc_agg(N, E, D)
    sc_deg = _make_sc_deg(N, E)
    degsum = _make_degsum(N, BM)
    mm = _make_mm(N, D, BM)
    gru_next = _make_gru(N, D, BM, False)
    gru_final = _make_gru(N, D, BM, True)

    hists = sc_deg(dst)
    degrow = degsum(hists.reshape(_NW, N))
    deg = degrow.reshape(N, 1)

    h = x
    m = mm(h, weight[0])
    for i in range(num_layers):
        aggp = sc_agg(m, src, dst)
        if i + 1 < num_layers:
            h, m = gru_next(aggp, deg, h, W_ihT, W_hhT, b_ih2, b_hh2,
                            weight[i + 1])
        else:
            out = gru_final(aggp, deg, h, W_ihT, W_hhT, b_ih2, b_hh2)
    return out


# speedup vs baseline: 8.6279x; 1.7432x over previous
import functools

import jax
import jax.numpy as jnp
from jax import lax
from jax.experimental import pallas as pl
from jax.experimental.pallas import tpu as pltpu
from jax.experimental.pallas import tpu_sc as plsc

_NC = 2
_NS = 16
_NW = _NC * _NS
_L = 16


def _sc_mesh():
    return plsc.VectorSubcoreMesh(core_axis_name="c", subcore_axis_name="s",
                                  num_cores=_NC, num_subcores=_NS)


@functools.cache
def _make_sc_agg(N, E, D):
    EPW = E // _NW
    K = 50
    NCH = EPW // K
    NSC = 5
    CH_L = NCH // NSC
    NP_L = CH_L // 2
    assert EPW % K == 0 and (EPW * _NW) == E
    assert NCH % NSC == 0 and CH_L % 8 == 0 and CH_L % 2 == 0
    CW = 10
    RPT = N // CW
    ZR = 40
    assert N % CW == 0 and RPT % ZR == 0 and ZR % 8 == 0 and K >= ZR

    @functools.partial(
        pl.kernel,
        out_type=jax.ShapeDtypeStruct((_NC, N, D), jnp.float32),
        mesh=_sc_mesh(),
        scratch_types=[
            pltpu.VMEM((CH_L, K), jnp.int32),
            pltpu.VMEM((CH_L, K), jnp.int32),
            pltpu.VMEM((K, D), jnp.float32),
            pltpu.VMEM((K, D), jnp.float32),
            pltpu.VMEM_SHARED((N, D), jnp.float32),
            pltpu.SemaphoreType.DMA,
            pltpu.SemaphoreType.DMA,
            pltpu.SemaphoreType.DMA,
            pltpu.SemaphoreType.DMA,
        ])
    def sc_agg(m_hbm, src_hbm, dst_hbm, agg_out, srcs, dsts, rows0, rows1,
               aggs, sg0, sg1, ss0, ss1):
        c = lax.axis_index("c")
        s = lax.axis_index("s")
        wid = c * _NS + s
        zv = jnp.zeros((_L,), jnp.float32)
        r0 = s * RPT

        @pl.when(s < CW)
        def _init():
            @pl.loop(0, ZR)
            def _zero_rows(i):
                for j in range(D // _L):
                    rows0[i, pl.ds(j * _L, _L)] = zv

            @pl.loop(0, RPT // ZR)
            def _zero_agg(t):
                pltpu.sync_copy(rows0.at[pl.ds(0, ZR)],
                                aggs.at[pl.ds(r0 + t * ZR, ZR)])

        plsc.subcore_barrier()

        def gather(t, buf, sem):
            pltpu.async_copy(m_hbm.at[srcs.at[t]], buf, sem)

        def gather_wait(buf, sem):
            pltpu.make_async_copy(m_hbm.at[srcs.at[0]], buf, sem).wait()

        def scatter(t, buf, sem):
            pltpu.async_copy(buf, aggs.at[dsts.at[t]], sem, add=True)

        def scatter_wait(t, buf, sem):
            pltpu.make_async_copy(buf, aggs.at[dsts.at[t]], sem).wait()

        @pl.loop(0, NSC)
        def _super(sc):
            pltpu.sync_copy(src_hbm.at[wid, pl.ds(sc * CH_L, CH_L)], srcs)
            pltpu.sync_copy(dst_hbm.at[wid, pl.ds(sc * CH_L, CH_L)], dsts)
            gather(0, rows0, sg0)

            @pl.loop(0, NP_L)
            def _pairs(p):
                t0 = 2 * p

                @pl.when(p > 0)
                def _w():
                    scatter_wait(t0 - 1, rows1, ss1)

                gather(t0 + 1, rows1, sg1)
                gather_wait(rows0, sg0)
                scatter(t0, rows0, ss0)
                gather_wait(rows1, sg1)
                scatter_wait(t0, rows0, ss0)

                @pl.when(p < NP_L - 1)
                def _g():
                    gather(t0 + 2, rows0, sg0)

                scatter(t0 + 1, rows1, ss1)

            scatter_wait(CH_L - 1, rows1, ss1)

        plsc.subcore_barrier()

        @pl.when(s < CW)
        def _copy_out():
            @pl.loop(0, RPT // ZR)
            def _copy_agg(t):
                rr = r0 + t * ZR
                pltpu.sync_copy(aggs.at[pl.ds(rr, ZR)], rows0.at[pl.ds(0, ZR)])
                pltpu.sync_copy(rows0.at[pl.ds(0, ZR)],
                                agg_out.at[c, pl.ds(rr, ZR)])

    return sc_agg


@functools.cache
def _make_sc_deg(N, E):
    EPW = E // _NW
    K = 80
    NCH = EPW // K
    assert EPW % K == 0 and N % _L == 0

    @functools.partial(
        pl.kernel,
        out_type=jax.ShapeDtypeStruct((_NW * N,), jnp.float32),
        mesh=_sc_mesh(),
        compiler_params=pltpu.CompilerParams(needs_layout_passes=False),
        scratch_types=[
            pltpu.VMEM((K,), jnp.int32),
            pltpu.VMEM((N,), jnp.float32),
        ])
    def sc_deg(dst_hbm, out_hbm, dstv, hist):
        c = lax.axis_index("c")
        s = lax.axis_index("s")
        wid = c * _NS + s
        zv = jnp.zeros((_L,), jnp.float32)
        ov = jnp.full((_L,), 1.0, jnp.float32)

        @pl.loop(0, N // _L)
        def _zero(i):
            hist[pl.ds(i * _L, _L)] = zv

        ebase = wid * EPW

        @pl.loop(0, NCH)
        def _edges(chk):
            off = ebase + chk * K
            pltpu.sync_copy(dst_hbm.at[pl.ds(off, K)], dstv)
            for j in range(K // _L):
                idx = dstv[pl.ds(j * _L, _L)]
                plsc.addupdate_scatter(hist, [idx], ov)

        pltpu.sync_copy(hist, out_hbm.at[pl.ds(wid * N, N)])

    return sc_deg


def _degsum_body(h_ref, o_ref):
    o_ref[...] = jnp.maximum(jnp.sum(h_ref[...], axis=0, keepdims=True), 1.0)


@functools.cache
def _make_degsum(N, BM):
    del BM
    return pl.pallas_call(
        _degsum_body,
        out_shape=jax.ShapeDtypeStruct((1, N), jnp.float32),
    )


def _mm_body(h_ref, w_ref, o_ref):
    o_ref[...] = jnp.dot(h_ref[...], w_ref[...],
                         preferred_element_type=jnp.float32)


@functools.cache
def _make_mm(N, D, BM):
    return pl.pallas_call(
        _mm_body,
        grid=(N // BM,),
        in_specs=[
            pl.BlockSpec((BM, D), lambda i: (i, 0)),
            pl.BlockSpec((D, D), lambda i: (0, 0)),
        ],
        out_specs=pl.BlockSpec((BM, D), lambda i: (i, 0)),
        out_shape=jax.ShapeDtypeStruct((N, D), jnp.float32),
    )


def _gru_cell(agg, h, wih_ref, whh_ref, bih_ref, bhh_ref):
    D = h.shape[-1]
    gi = jnp.dot(agg, wih_ref[...], preferred_element_type=jnp.float32)
    gi = gi + bih_ref[...]
    gh = jnp.dot(h, whh_ref[...], preferred_element_type=jnp.float32)
    gh = gh + bhh_ref[...]
    r = jax.nn.sigmoid(gi[:, :D] + gh[:, :D])
    z = jax.nn.sigmoid(gi[:, D:2 * D] + gh[:, D:2 * D])
    n = jnp.tanh(gi[:, 2 * D:] + r * gh[:, 2 * D:])
    return (1.0 - z) * n + z * h


def _gru_next_body(aggp_ref, deg_ref, h_ref, wih_ref, whh_ref, bih_ref,
                   bhh_ref, wnext_ref, hout_ref, mout_ref):
    agg = (aggp_ref[0] + aggp_ref[1]) / deg_ref[...]
    hn = _gru_cell(agg, h_ref[...], wih_ref, whh_ref, bih_ref, bhh_ref)
    hout_ref[...] = hn
    mout_ref[...] = jnp.dot(hn, wnext_ref[...],
                            preferred_element_type=jnp.float32)


def _gru_final_body(aggp_ref, deg_ref, h_ref, wih_ref, whh_ref, bih_ref,
                    bhh_ref, out_ref):
    agg = (aggp_ref[0] + aggp_ref[1]) / deg_ref[...]
    hn = _gru_cell(agg, h_ref[...], wih_ref, whh_ref, bih_ref, bhh_ref)
    out_ref[...] = jnp.tanh(jnp.mean(hn, axis=1, keepdims=True))


@functools.cache
def _make_gru(N, D, BM, final):
    in_specs = [
        pl.BlockSpec((2, BM, D), lambda i: (0, i, 0)),
        pl.BlockSpec((BM, 1), lambda i: (i, 0)),
        pl.BlockSpec((BM, D), lambda i: (i, 0)),
        pl.BlockSpec((D, 3 * D), lambda i: (0, 0)),
        pl.BlockSpec((D, 3 * D), lambda i: (0, 0)),
        pl.BlockSpec((1, 3 * D), lambda i: (0, 0)),
        pl.BlockSpec((1, 3 * D), lambda i: (0, 0)),
    ]
    if final:
        return pl.pallas_call(
            _gru_final_body,
            grid=(N // BM,),
            in_specs=in_specs,
            out_specs=pl.BlockSpec((BM, 1), lambda i: (i, 0)),
            out_shape=jax.ShapeDtypeStruct((N, 1), jnp.float32),
        )
    in_specs.append(pl.BlockSpec((D, D), lambda i: (0, 0)))
    return pl.pallas_call(
        _gru_next_body,
        grid=(N // BM,),
        in_specs=in_specs,
        out_specs=[pl.BlockSpec((BM, D), lambda i: (i, 0)),
                   pl.BlockSpec((BM, D), lambda i: (i, 0))],
        out_shape=[jax.ShapeDtypeStruct((N, D), jnp.float32),
                   jax.ShapeDtypeStruct((N, D), jnp.float32)],
    )


def kernel(x, edge_index, weight, W_ih, W_hh, b_ih, b_hh):
    N, D = x.shape
    E = edge_index.shape[1]
    num_layers = weight.shape[0]
    BM = 2000

    src = edge_index[0]
    dst = edge_index[1]
    K = 50
    NCH = E // _NW // K
    src3 = src.reshape(_NW, NCH, K)
    dst3 = dst.reshape(_NW, NCH, K)
    W_ihT = W_ih.T
    W_hhT = W_hh.T
    b_ih2 = b_ih.reshape(1, -1)
    b_hh2 = b_hh.reshape(1, -1)

    sc_agg = _make_sc_agg(N, E, D)
    sc_deg = _make_sc_deg(N, E)
    degsum = _make_degsum(N, BM)
    mm = _make_mm(N, D, BM)
    gru_next = _make_gru(N, D, BM, False)
    gru_final = _make_gru(N, D, BM, True)

    hists = sc_deg(dst)
    degrow = degsum(hists.reshape(_NW, N))
    deg = degrow.reshape(N, 1)

    h = x
    m = mm(h, weight[0])
    for i in range(num_layers):
        aggp = sc_agg(m, src3, dst3)
        if i + 1 < num_layers:
            h, m = gru_next(aggp, deg, h, W_ihT, W_hhT, b_ih2, b_hh2,
                            weight[i + 1])
        else:
            out = gru_final(aggp, deg, h, W_ihT, W_hhT, b_ih2, b_hh2)
    return out


# speedup vs baseline: 9.2693x; 1.0743x over previous
import functools

import jax
import jax.numpy as jnp
from jax import lax
from jax.experimental import pallas as pl
from jax.experimental.pallas import tpu as pltpu
from jax.experimental.pallas import tpu_sc as plsc

_NC = 2
_NS = 16
_NW = _NC * _NS
_L = 16


def _sc_mesh():
    return plsc.VectorSubcoreMesh(core_axis_name="c", subcore_axis_name="s",
                                  num_cores=_NC, num_subcores=_NS)


@functools.cache
def _make_sc_agg(N, E, D):
    EPW = E // _NW
    K = 50
    NCH = EPW // K
    NSC = 5
    CH_L = NCH // NSC
    NP_L = CH_L // 2
    assert EPW % K == 0 and (EPW * _NW) == E
    assert NCH % NSC == 0 and CH_L % 8 == 0 and CH_L % 2 == 0
    CW = 10
    RPT = N // CW
    ZR = 40
    assert N % CW == 0 and RPT % ZR == 0 and ZR % 8 == 0 and K >= ZR

    @functools.partial(
        pl.kernel,
        out_type=jax.ShapeDtypeStruct((_NC, N, D), jnp.float32),
        mesh=_sc_mesh(),
        scratch_types=[
            pltpu.VMEM((CH_L, K), jnp.int32),
            pltpu.VMEM((CH_L, K), jnp.int32),
            pltpu.VMEM((K, D), jnp.float32),
            pltpu.VMEM((K, D), jnp.float32),
            pltpu.VMEM_SHARED((N, D), jnp.float32),
            pltpu.SemaphoreType.DMA,
            pltpu.SemaphoreType.DMA,
            pltpu.SemaphoreType.DMA,
            pltpu.SemaphoreType.DMA,
        ])
    def sc_agg(m_hbm, src_hbm, dst_hbm, agg_out, srcs, dsts, rows0, rows1,
               aggs, sg0, sg1, ss0, ss1):
        c = lax.axis_index("c")
        s = lax.axis_index("s")
        wid = c * _NS + s
        zv = jnp.zeros((_L,), jnp.float32)
        r0 = s * RPT

        @pl.when(s < CW)
        def _init():
            @pl.loop(0, ZR)
            def _zero_rows(i):
                for j in range(D // _L):
                    rows0[i, pl.ds(j * _L, _L)] = zv

            @pl.loop(0, RPT // ZR)
            def _zero_agg(t):
                pltpu.sync_copy(rows0.at[pl.ds(0, ZR)],
                                aggs.at[pl.ds(r0 + t * ZR, ZR)])

        plsc.subcore_barrier()

        def gather(t, buf, sem):
            pltpu.async_copy(m_hbm.at[srcs.at[t]], buf, sem)

        def gather_wait(buf, sem):
            pltpu.make_async_copy(m_hbm.at[srcs.at[0]], buf, sem).wait()

        def scatter(t, buf, sem):
            pltpu.async_copy(buf, aggs.at[dsts.at[t]], sem, add=True)

        def scatter_wait(t, buf, sem):
            pltpu.make_async_copy(buf, aggs.at[dsts.at[t]], sem).wait()

        @pl.loop(0, NSC)
        def _super(sc):
            pltpu.sync_copy(src_hbm.at[wid, pl.ds(sc * CH_L, CH_L)], srcs)
            pltpu.sync_copy(dst_hbm.at[wid, pl.ds(sc * CH_L, CH_L)], dsts)
            gather(0, rows0, sg0)

            @pl.loop(0, NP_L)
            def _pairs(p):
                t0 = 2 * p

                @pl.when(p > 0)
                def _w():
                    scatter_wait(t0 - 1, rows1, ss1)

                gather(t0 + 1, rows1, sg1)
                gather_wait(rows0, sg0)
                scatter(t0, rows0, ss0)
                gather_wait(rows1, sg1)
                scatter_wait(t0, rows0, ss0)

                @pl.when(p < NP_L - 1)
                def _g():
                    gather(t0 + 2, rows0, sg0)

                scatter(t0 + 1, rows1, ss1)

            scatter_wait(CH_L - 1, rows1, ss1)

        plsc.subcore_barrier()

        @pl.when(s < CW)
        def _copy_out():
            @pl.loop(0, RPT // ZR)
            def _copy_agg(t):
                rr = r0 + t * ZR
                pltpu.sync_copy(aggs.at[pl.ds(rr, ZR)], rows0.at[pl.ds(0, ZR)])
                pltpu.sync_copy(rows0.at[pl.ds(0, ZR)],
                                agg_out.at[c, pl.ds(rr, ZR)])

    return sc_agg


@functools.cache
def _make_sc_deg(N, E):
    EPW = E // _NW
    assert EPW % _L == 0 and N % _L == 0

    @functools.partial(
        pl.kernel,
        out_type=jax.ShapeDtypeStruct((_NW * N,), jnp.float32),
        mesh=_sc_mesh(),
        compiler_params=pltpu.CompilerParams(needs_layout_passes=False),
        scratch_types=[
            pltpu.VMEM((EPW,), jnp.int32),
            pltpu.VMEM((N,), jnp.float32),
        ])
    def sc_deg(dst_hbm, out_hbm, dstv, hist):
        c = lax.axis_index("c")
        s = lax.axis_index("s")
        wid = c * _NS + s
        zv = jnp.zeros((_L,), jnp.float32)
        ov = jnp.full((_L,), 1.0, jnp.float32)

        pltpu.sync_copy(dst_hbm.at[wid], dstv)

        @pl.loop(0, N // _L)
        def _zero(i):
            hist[pl.ds(i * _L, _L)] = zv

        @pl.loop(0, EPW // _L)
        def _edges(j):
            idx = dstv[pl.ds(j * _L, _L)]
            plsc.addupdate_scatter(hist, [idx], ov)

        pltpu.sync_copy(hist, out_hbm.at[pl.ds(wid * N, N)])

    return sc_deg


def _degsum_body(h_ref, o_ref):
    o_ref[...] = jnp.maximum(jnp.sum(h_ref[...], axis=0, keepdims=True), 1.0)


@functools.cache
def _make_degsum(N, BM):
    del BM
    return pl.pallas_call(
        _degsum_body,
        out_shape=jax.ShapeDtypeStruct((1, N), jnp.float32),
    )


def _mm_body(h_ref, w_ref, o_ref):
    o_ref[...] = jnp.dot(h_ref[...], w_ref[...],
                         preferred_element_type=jnp.float32)


@functools.cache
def _make_mm(N, D, BM):
    return pl.pallas_call(
        _mm_body,
        grid=(N // BM,),
        in_specs=[
            pl.BlockSpec((BM, D), lambda i: (i, 0)),
            pl.BlockSpec((D, D), lambda i: (0, 0)),
        ],
        out_specs=pl.BlockSpec((BM, D), lambda i: (i, 0)),
        out_shape=jax.ShapeDtypeStruct((N, D), jnp.float32),
    )


def _gru_cell(agg, h, wih_ref, whh_ref, bih_ref, bhh_ref):
    D = h.shape[-1]
    gi = jnp.dot(agg, wih_ref[...], preferred_element_type=jnp.float32)
    gi = gi + bih_ref[...]
    gh = jnp.dot(h, whh_ref[...], preferred_element_type=jnp.float32)
    gh = gh + bhh_ref[...]
    r = jax.nn.sigmoid(gi[:, :D] + gh[:, :D])
    z = jax.nn.sigmoid(gi[:, D:2 * D] + gh[:, D:2 * D])
    n = jnp.tanh(gi[:, 2 * D:] + r * gh[:, 2 * D:])
    return (1.0 - z) * n + z * h


def _gru_next_body(aggp_ref, deg_ref, h_ref, wih_ref, whh_ref, bih_ref,
                   bhh_ref, wnext_ref, hout_ref, mout_ref):
    agg = (aggp_ref[0] + aggp_ref[1]) / deg_ref[...]
    hn = _gru_cell(agg, h_ref[...], wih_ref, whh_ref, bih_ref, bhh_ref)
    hout_ref[...] = hn
    mout_ref[...] = jnp.dot(hn, wnext_ref[...],
                            preferred_element_type=jnp.float32)


def _gru_final_body(aggp_ref, deg_ref, h_ref, wih_ref, whh_ref, bih_ref,
                    bhh_ref, out_ref):
    agg = (aggp_ref[0] + aggp_ref[1]) / deg_ref[...]
    hn = _gru_cell(agg, h_ref[...], wih_ref, whh_ref, bih_ref, bhh_ref)
    out_ref[...] = jnp.tanh(jnp.mean(hn, axis=1, keepdims=True))


@functools.cache
def _make_gru(N, D, BM, final):
    in_specs = [
        pl.BlockSpec((2, BM, D), lambda i: (0, i, 0)),
        pl.BlockSpec((BM, 1), lambda i: (i, 0)),
        pl.BlockSpec((BM, D), lambda i: (i, 0)),
        pl.BlockSpec((D, 3 * D), lambda i: (0, 0)),
        pl.BlockSpec((D, 3 * D), lambda i: (0, 0)),
        pl.BlockSpec((1, 3 * D), lambda i: (0, 0)),
        pl.BlockSpec((1, 3 * D), lambda i: (0, 0)),
    ]
    if final:
        return pl.pallas_call(
            _gru_final_body,
            grid=(N // BM,),
            in_specs=in_specs,
            out_specs=pl.BlockSpec((BM, 1), lambda i: (i, 0)),
            out_shape=jax.ShapeDtypeStruct((N, 1), jnp.float32),
        )
    in_specs.append(pl.BlockSpec((D, D), lambda i: (0, 0)))
    return pl.pallas_call(
        _gru_next_body,
        grid=(N // BM,),
        in_specs=in_specs,
        out_specs=[pl.BlockSpec((BM, D), lambda i: (i, 0)),
                   pl.BlockSpec((BM, D), lambda i: (i, 0))],
        out_shape=[jax.ShapeDtypeStruct((N, D), jnp.float32),
                   jax.ShapeDtypeStruct((N, D), jnp.float32)],
    )


def kernel(x, edge_index, weight, W_ih, W_hh, b_ih, b_hh):
    N, D = x.shape
    E = edge_index.shape[1]
    num_layers = weight.shape[0]
    BM = 2000

    src = edge_index[0]
    dst = edge_index[1]
    K = 50
    NCH = E // _NW // K
    src3 = src.reshape(_NW, NCH, K)
    dst3 = dst.reshape(_NW, NCH, K)
    W_ihT = W_ih.T
    W_hhT = W_hh.T
    b_ih2 = b_ih.reshape(1, -1)
    b_hh2 = b_hh.reshape(1, -1)

    sc_agg = _make_sc_agg(N, E, D)
    sc_deg = _make_sc_deg(N, E)
    degsum = _make_degsum(N, BM)
    mm = _make_mm(N, D, BM)
    gru_next = _make_gru(N, D, BM, False)
    gru_final = _make_gru(N, D, BM, True)

    hists = sc_deg(dst.reshape(_NW, E // _NW))
    degrow = degsum(hists.reshape(_NW, N))
    deg = degrow.reshape(N, 1)

    h = x
    m = mm(h, weight[0])
    for i in range(num_layers):
        aggp = sc_agg(m, src3, dst3)
        if i + 1 < num_layers:
            h, m = gru_next(aggp, deg, h, W_ihT, W_hhT, b_ih2, b_hh2,
                            weight[i + 1])
        else:
            out = gru_final(aggp, deg, h, W_ihT, W_hhT, b_ih2, b_hh2)
    return out


# speedup vs baseline: 9.2752x; 1.0006x over previous
import functools

import jax
import jax.numpy as jnp
from jax import lax
from jax.experimental import pallas as pl
from jax.experimental.pallas import tpu as pltpu
from jax.experimental.pallas import tpu_sc as plsc

_NC = 2
_NS = 16
_NW = _NC * _NS
_L = 16


def _sc_mesh():
    return plsc.VectorSubcoreMesh(core_axis_name="c", subcore_axis_name="s",
                                  num_cores=_NC, num_subcores=_NS)


@functools.cache
def _make_sc_agg(N, E, D):
    EPW = E // _NW
    K = 50
    NCH = EPW // K
    NSC = 5
    CH_L = NCH // NSC
    NP_L = CH_L // 2
    assert EPW % K == 0 and (EPW * _NW) == E
    assert NCH % NSC == 0 and CH_L % 8 == 0 and CH_L % 2 == 0
    CW = 10
    RPT = N // CW
    ZR = 40
    assert N % CW == 0 and RPT % ZR == 0 and ZR % 8 == 0 and K >= ZR

    @functools.partial(
        pl.kernel,
        out_type=jax.ShapeDtypeStruct((_NC, N, D), jnp.float32),
        mesh=_sc_mesh(),
        scratch_types=[
            pltpu.VMEM((CH_L, K), jnp.int32),
            pltpu.VMEM((CH_L, K), jnp.int32),
            pltpu.VMEM((K, D), jnp.float32),
            pltpu.VMEM((K, D), jnp.float32),
            pltpu.VMEM_SHARED((N, D), jnp.float32),
            pltpu.SemaphoreType.DMA,
            pltpu.SemaphoreType.DMA,
            pltpu.SemaphoreType.DMA,
            pltpu.SemaphoreType.DMA,
        ])
    def sc_agg(m_hbm, src_hbm, dst_hbm, agg_out, srcs, dsts, rows0, rows1,
               aggs, sg0, sg1, ss0, ss1):
        c = lax.axis_index("c")
        s = lax.axis_index("s")
        wid = c * _NS + s
        zv = jnp.zeros((_L,), jnp.float32)
        r0 = s * RPT

        @pl.when(s < CW)
        def _init():
            @pl.loop(0, ZR)
            def _zero_rows(i):
                for j in range(D // _L):
                    rows0[i, pl.ds(j * _L, _L)] = zv

            @pl.loop(0, RPT // ZR)
            def _zero_agg(t):
                pltpu.sync_copy(rows0.at[pl.ds(0, ZR)],
                                aggs.at[pl.ds(r0 + t * ZR, ZR)])

        plsc.subcore_barrier()

        def gather(t, buf, sem):
            pltpu.async_copy(m_hbm.at[srcs.at[t]], buf, sem)

        def gather_wait(buf, sem):
            pltpu.make_async_copy(m_hbm.at[srcs.at[0]], buf, sem).wait()

        def scatter(t, buf, sem):
            pltpu.async_copy(buf, aggs.at[dsts.at[t]], sem, add=True)

        def scatter_wait(t, buf, sem):
            pltpu.make_async_copy(buf, aggs.at[dsts.at[t]], sem).wait()

        @pl.loop(0, NSC)
        def _super(sc):
            pltpu.sync_copy(src_hbm.at[wid, pl.ds(sc * CH_L, CH_L)], srcs)
            pltpu.sync_copy(dst_hbm.at[wid, pl.ds(sc * CH_L, CH_L)], dsts)
            gather(0, rows0, sg0)

            @pl.loop(0, NP_L)
            def _pairs(p):
                t0 = 2 * p

                @pl.when(p > 0)
                def _w():
                    scatter_wait(t0 - 1, rows1, ss1)

                gather(t0 + 1, rows1, sg1)
                gather_wait(rows0, sg0)
                scatter(t0, rows0, ss0)
                gather_wait(rows1, sg1)
                scatter_wait(t0, rows0, ss0)

                @pl.when(p < NP_L - 1)
                def _g():
                    gather(t0 + 2, rows0, sg0)

                scatter(t0 + 1, rows1, ss1)

            scatter_wait(CH_L - 1, rows1, ss1)

        plsc.subcore_barrier()

        @pl.when(s < CW)
        def _copy_out():
            @pl.loop(0, RPT // ZR)
            def _copy_agg(t):
                rr = r0 + t * ZR
                pltpu.sync_copy(aggs.at[pl.ds(rr, ZR)], rows0.at[pl.ds(0, ZR)])
                pltpu.sync_copy(rows0.at[pl.ds(0, ZR)],
                                agg_out.at[c, pl.ds(rr, ZR)])

    return sc_agg


@functools.cache
def _make_sc_deg(N, E):
    EPW = E // _NW
    assert EPW % _L == 0 and N % _L == 0

    @functools.partial(
        pl.kernel,
        out_type=jax.ShapeDtypeStruct((_NW * N,), jnp.float32),
        mesh=_sc_mesh(),
        compiler_params=pltpu.CompilerParams(needs_layout_passes=False),
        scratch_types=[
            pltpu.VMEM((EPW,), jnp.int32),
            pltpu.VMEM((N,), jnp.float32),
        ])
    def sc_deg(dst_hbm, out_hbm, dstv, hist):
        c = lax.axis_index("c")
        s = lax.axis_index("s")
        wid = c * _NS + s
        zv = jnp.zeros((_L,), jnp.float32)
        ov = jnp.full((_L,), 1.0, jnp.float32)

        pltpu.sync_copy(dst_hbm.at[wid], dstv)

        @pl.loop(0, N // _L)
        def _zero(i):
            hist[pl.ds(i * _L, _L)] = zv

        @pl.loop(0, EPW // _L)
        def _edges(j):
            idx = dstv[pl.ds(j * _L, _L)]
            plsc.addupdate_scatter(hist, [idx], ov)

        pltpu.sync_copy(hist, out_hbm.at[pl.ds(wid * N, N)])

    return sc_deg


def _degsum_body(h_ref, o_ref):
    dsum = jnp.maximum(jnp.sum(h_ref[...], axis=0, keepdims=True), 1.0)
    o_ref[...] = dsum.reshape(o_ref.shape)


@functools.cache
def _make_degsum(N, BM):
    del BM
    return pl.pallas_call(
        _degsum_body,
        out_shape=jax.ShapeDtypeStruct((N, 1), jnp.float32),
    )


def _mm_body(h_ref, w_ref, o_ref):
    o_ref[...] = jnp.dot(h_ref[...], w_ref[...],
                         preferred_element_type=jnp.float32)


@functools.cache
def _make_mm(N, D, BM):
    return pl.pallas_call(
        _mm_body,
        grid=(N // BM,),
        in_specs=[
            pl.BlockSpec((BM, D), lambda i: (i, 0)),
            pl.BlockSpec((D, D), lambda i: (0, 0)),
        ],
        out_specs=pl.BlockSpec((BM, D), lambda i: (i, 0)),
        out_shape=jax.ShapeDtypeStruct((N, D), jnp.float32),
    )


def _gru_cell(agg, h, wih_ref, whh_ref, bih_ref, bhh_ref):
    D = h.shape[-1]
    gi = jnp.dot(agg, wih_ref[...], preferred_element_type=jnp.float32)
    gi = gi + bih_ref[...]
    gh = jnp.dot(h, whh_ref[...], preferred_element_type=jnp.float32)
    gh = gh + bhh_ref[...]
    r = jax.nn.sigmoid(gi[:, :D] + gh[:, :D])
    z = jax.nn.sigmoid(gi[:, D:2 * D] + gh[:, D:2 * D])
    n = jnp.tanh(gi[:, 2 * D:] + r * gh[:, 2 * D:])
    return (1.0 - z) * n + z * h


def _gru_next_body(aggp_ref, deg_ref, h_ref, wih_ref, whh_ref, bih_ref,
                   bhh_ref, wnext_ref, hout_ref, mout_ref):
    agg = (aggp_ref[0] + aggp_ref[1]) / deg_ref[...]
    hn = _gru_cell(agg, h_ref[...], wih_ref, whh_ref, bih_ref, bhh_ref)
    hout_ref[...] = hn
    mout_ref[...] = jnp.dot(hn, wnext_ref[...],
                            preferred_element_type=jnp.float32)


def _gru_final_body(aggp_ref, deg_ref, h_ref, wih_ref, whh_ref, bih_ref,
                    bhh_ref, out_ref):
    agg = (aggp_ref[0] + aggp_ref[1]) / deg_ref[...]
    hn = _gru_cell(agg, h_ref[...], wih_ref, whh_ref, bih_ref, bhh_ref)
    out_ref[...] = jnp.tanh(jnp.mean(hn, axis=1, keepdims=True))


@functools.cache
def _make_gru(N, D, BM, final):
    in_specs = [
        pl.BlockSpec((2, BM, D), lambda i: (0, i, 0)),
        pl.BlockSpec((BM, 1), lambda i: (i, 0)),
        pl.BlockSpec((BM, D), lambda i: (i, 0)),
        pl.BlockSpec((D, 3 * D), lambda i: (0, 0)),
        pl.BlockSpec((D, 3 * D), lambda i: (0, 0)),
        pl.BlockSpec((1, 3 * D), lambda i: (0, 0)),
        pl.BlockSpec((1, 3 * D), lambda i: (0, 0)),
    ]
    if final:
        return pl.pallas_call(
            _gru_final_body,
            grid=(N // BM,),
            in_specs=in_specs,
            out_specs=pl.BlockSpec((BM, 1), lambda i: (i, 0)),
            out_shape=jax.ShapeDtypeStruct((N, 1), jnp.float32),
        )
    in_specs.append(pl.BlockSpec((D, D), lambda i: (0, 0)))
    return pl.pallas_call(
        _gru_next_body,
        grid=(N // BM,),
        in_specs=in_specs,
        out_specs=[pl.BlockSpec((BM, D), lambda i: (i, 0)),
                   pl.BlockSpec((BM, D), lambda i: (i, 0))],
        out_shape=[jax.ShapeDtypeStruct((N, D), jnp.float32),
                   jax.ShapeDtypeStruct((N, D), jnp.float32)],
    )


def kernel(x, edge_index, weight, W_ih, W_hh, b_ih, b_hh):
    N, D = x.shape
    E = edge_index.shape[1]
    num_layers = weight.shape[0]
    BM = 2000

    src = edge_index[0]
    dst = edge_index[1]
    K = 50
    NCH = E // _NW // K
    src3 = src.reshape(_NW, NCH, K)
    dst3 = dst.reshape(_NW, NCH, K)
    W_ihT = W_ih.T
    W_hhT = W_hh.T
    b_ih2 = b_ih.reshape(1, -1)
    b_hh2 = b_hh.reshape(1, -1)

    sc_agg = _make_sc_agg(N, E, D)
    sc_deg = _make_sc_deg(N, E)
    degsum = _make_degsum(N, BM)
    mm = _make_mm(N, D, BM)
    gru_next = _make_gru(N, D, BM, False)
    gru_final = _make_gru(N, D, BM, True)

    hists = sc_deg(dst.reshape(_NW, E // _NW))
    deg = degsum(hists.reshape(_NW, N))

    h = x
    m = mm(h, weight[0])
    for i in range(num_layers):
        aggp = sc_agg(m, src3, dst3)
        if i + 1 < num_layers:
            h, m = gru_next(aggp, deg, h, W_ihT, W_hhT, b_ih2, b_hh2,
                            weight[i + 1])
        else:
            out = gru_final(aggp, deg, h, W_ihT, W_hhT, b_ih2, b_hh2)
    return out
